# Initial kernel scaffold; baseline (speedup 1.0000x reference)
import jax
import jax.numpy as jnp
from jax.experimental import pallas as pl


def kernel(x, Wr, w1, b1, w2, b2):
    raise NotImplementedError("write your pallas kernel here")


# speedup vs baseline: 1.6762x; 1.6762x over previous
import functools

import jax
import jax.numpy as jnp
from jax import lax
from jax.experimental import pallas as pl
from jax.experimental.pallas import tpu as pltpu

_E = 8
_K = 2
_TILE = 256

_INTERPRET = False


def _router_body(x_ref, wr_ref, gates_ref, idx_ref):
    logits = jnp.dot(x_ref[...], wr_ref[...], preferred_element_type=jnp.float32)
    ei = lax.broadcasted_iota(jnp.int32, logits.shape, 1)
    m1 = jnp.max(logits, axis=1, keepdims=True)
    i1 = jnp.min(jnp.where(logits == m1, ei, _E), axis=1, keepdims=True)
    masked = jnp.where(ei == i1, -jnp.inf, logits)
    m2 = jnp.max(masked, axis=1, keepdims=True)
    i2 = jnp.min(jnp.where(masked == m2, ei, _E), axis=1, keepdims=True)
    z = jnp.exp(m2 - m1)
    g1 = 1.0 / (1.0 + z)
    gates_ref[...] = jnp.concatenate([g1, 1.0 - g1], axis=1)
    idx_ref[...] = jnp.concatenate([i1, i2], axis=1).astype(jnp.int32)


def _router(flat_x, Wr):
    n = flat_x.shape[0]
    return pl.pallas_call(
        _router_body,
        out_shape=[
            jax.ShapeDtypeStruct((n, _K), jnp.float32),
            jax.ShapeDtypeStruct((n, _K), jnp.int32),
        ],
        interpret=_INTERPRET,
    )(flat_x, Wr)


def _build_dispatch(idx, gates, n_blocks):
    s = idx.shape[0] * _K
    pad_rows = n_blocks * _TILE
    ef = idx.reshape(-1)
    gf = gates.reshape(-1)
    oh = (ef[:, None] == jnp.arange(_E, dtype=jnp.int32)[None, :]).astype(jnp.int32)
    counts = jnp.sum(oh, axis=0)
    rank = jnp.sum(jnp.cumsum(oh, axis=0) * oh, axis=1) - 1
    nblk = (counts + _TILE - 1) // _TILE
    cb = jnp.cumsum(nblk)
    off = (cb - nblk) * _TILE
    pos = off[ef] + rank
    tok = jnp.arange(s, dtype=jnp.int32) // _K
    row_token = jnp.zeros((pad_rows,), jnp.int32).at[pos].set(tok)
    row_gate = jnp.zeros((pad_rows,), jnp.float32).at[pos].set(gf)
    gi = jnp.arange(n_blocks, dtype=jnp.int32)
    block_expert = jnp.sum(gi[:, None] >= cb[None, :], axis=1).astype(jnp.int32)
    sp = jnp.stack([jnp.minimum(block_expert, _E - 1), block_expert])
    return row_token, row_gate, pos.reshape(-1, _K), sp


def _ffn_body(sp_ref, xs_ref, gate_ref, w1_ref, b1_ref, w2_ref, b2_ref, ys_ref):
    ff = w2_ref.shape[1]

    @pl.when(sp_ref[1, pl.program_id(0)] < _E)
    def _():
        u = jnp.dot(xs_ref[...], w1_ref[0], preferred_element_type=jnp.float32)
        u = u + b1_ref[0]
        ua = u[:, :ff]
        ub = u[:, ff:]
        h = ua * jax.nn.sigmoid(ua) * ub
        y = jnp.dot(h, w2_ref[0], preferred_element_type=jnp.float32)
        y = y + b2_ref[0]
        ys_ref[...] = y * gate_ref[0, 0][:, None]


def _grouped_ffn(xs, row_gate, w1, b1, w2, b2, sp, n_blocks):
    d = xs.shape[1]
    ff = w2.shape[1]
    pad_rows = n_blocks * _TILE
    grid_spec = pltpu.PrefetchScalarGridSpec(
        num_scalar_prefetch=1,
        grid=(n_blocks,),
        in_specs=[
            pl.BlockSpec((_TILE, d), lambda g, sp: (g, 0)),
            pl.BlockSpec((1, 1, _TILE), lambda g, sp: (g, 0, 0)),
            pl.BlockSpec((1, d, 2 * ff), lambda g, sp: (sp[0, g], 0, 0)),
            pl.BlockSpec((1, 1, 2 * ff), lambda g, sp: (sp[0, g], 0, 0)),
            pl.BlockSpec((1, ff, d), lambda g, sp: (sp[0, g], 0, 0)),
            pl.BlockSpec((1, 1, d), lambda g, sp: (sp[0, g], 0, 0)),
        ],
        out_specs=pl.BlockSpec((_TILE, d), lambda g, sp: (g, 0)),
    )
    return pl.pallas_call(
        _ffn_body,
        grid_spec=grid_spec,
        out_shape=jax.ShapeDtypeStruct((pad_rows, d), jnp.float32),
        interpret=_INTERPRET,
    )(
        sp,
        xs,
        row_gate.reshape(n_blocks, 1, _TILE),
        w1,
        b1.reshape(_E, 1, 2 * ff),
        w2,
        b2.reshape(_E, 1, d),
    )


def kernel(x, Wr, w1, b1, w2, b2):
    bx, tx, d = x.shape
    n = bx * tx
    s = n * _K
    n_blocks = s // _TILE + _E
    flat_x = x.reshape(n, d)

    gates, idx = _router(flat_x, Wr)
    row_token, row_gate, pos, sp = _build_dispatch(idx, gates, n_blocks)
    xs = flat_x[row_token]
    ys = _grouped_ffn(xs, row_gate, w1, b1, w2, b2, sp, n_blocks)
    out = ys[pos[:, 0]] + ys[pos[:, 1]]
    return out.reshape(bx, tx, d)


# speedup vs baseline: 1.6790x; 1.0017x over previous
import functools

import jax
import jax.numpy as jnp
from jax import lax
from jax.experimental import pallas as pl
from jax.experimental.pallas import tpu as pltpu

_E = 8
_K = 2
_TILE = 256

_INTERPRET = False


def _router_body(x_ref, wr_ref, gates_ref, idx_ref):
    logits = jnp.dot(x_ref[...], wr_ref[...], preferred_element_type=jnp.float32)
    ei = lax.broadcasted_iota(jnp.int32, logits.shape, 1)
    m1 = jnp.max(logits, axis=1, keepdims=True)
    i1 = jnp.min(jnp.where(logits == m1, ei, _E), axis=1, keepdims=True)
    masked = jnp.where(ei == i1, -jnp.inf, logits)
    m2 = jnp.max(masked, axis=1, keepdims=True)
    i2 = jnp.min(jnp.where(masked == m2, ei, _E), axis=1, keepdims=True)
    z = jnp.exp(m2 - m1)
    g1 = 1.0 / (1.0 + z)
    gates_ref[...] = jnp.concatenate([g1, 1.0 - g1], axis=1)
    idx_ref[...] = jnp.concatenate([i1, i2], axis=1).astype(jnp.int32)


def _router(flat_x, Wr):
    n = flat_x.shape[0]
    return pl.pallas_call(
        _router_body,
        out_shape=[
            jax.ShapeDtypeStruct((n, _K), jnp.float32),
            jax.ShapeDtypeStruct((n, _K), jnp.int32),
        ],
        interpret=_INTERPRET,
    )(flat_x, Wr)


def _build_dispatch(idx, gates, n_blocks):
    s = idx.shape[0] * _K
    pad_rows = n_blocks * _TILE
    ef = idx.reshape(-1)
    gf = gates.reshape(-1)
    oh = (ef[:, None] == jnp.arange(_E, dtype=jnp.int32)[None, :]).astype(jnp.int32)
    counts = jnp.sum(oh, axis=0)
    rank = jnp.sum(jnp.cumsum(oh, axis=0) * oh, axis=1) - 1
    nblk = (counts + _TILE - 1) // _TILE
    cb = jnp.cumsum(nblk)
    off = (cb - nblk) * _TILE
    pos = off[ef] + rank
    tok = jnp.arange(s, dtype=jnp.int32) // _K
    row_token = jnp.zeros((pad_rows,), jnp.int32).at[pos].set(tok)
    row_gate = jnp.zeros((pad_rows,), jnp.float32).at[pos].set(gf)
    gi = jnp.arange(n_blocks, dtype=jnp.int32)
    block_expert = jnp.sum(gi[:, None] >= cb[None, :], axis=1).astype(jnp.int32)
    sp = jnp.stack([jnp.minimum(block_expert, _E - 1), block_expert])
    return row_token, row_gate, pos.reshape(-1, _K), sp


def _ffn_body(sp_ref, xs_ref, gate_ref, w1_ref, b1_ref, w2_ref, b2_ref, ys_ref):
    ff = w2_ref.shape[1]

    @pl.when(sp_ref[1, pl.program_id(0)] < _E)
    def _():
        xb = xs_ref[...].astype(jnp.bfloat16)
        u = jnp.dot(xb, w1_ref[0].astype(jnp.bfloat16),
                    preferred_element_type=jnp.float32)
        u = u + b1_ref[0]
        ua = u[:, :ff]
        ub = u[:, ff:]
        h = ua * jax.nn.sigmoid(ua) * ub
        y = jnp.dot(h.astype(jnp.bfloat16), w2_ref[0].astype(jnp.bfloat16),
                    preferred_element_type=jnp.float32)
        y = y + b2_ref[0]
        ys_ref[...] = y * gate_ref[0, 0][:, None]


def _grouped_ffn(xs, row_gate, w1, b1, w2, b2, sp, n_blocks):
    d = xs.shape[1]
    ff = w2.shape[1]
    pad_rows = n_blocks * _TILE
    grid_spec = pltpu.PrefetchScalarGridSpec(
        num_scalar_prefetch=1,
        grid=(n_blocks,),
        in_specs=[
            pl.BlockSpec((_TILE, d), lambda g, sp: (g, 0)),
            pl.BlockSpec((1, 1, _TILE), lambda g, sp: (g, 0, 0)),
            pl.BlockSpec((1, d, 2 * ff), lambda g, sp: (sp[0, g], 0, 0)),
            pl.BlockSpec((1, 1, 2 * ff), lambda g, sp: (sp[0, g], 0, 0)),
            pl.BlockSpec((1, ff, d), lambda g, sp: (sp[0, g], 0, 0)),
            pl.BlockSpec((1, 1, d), lambda g, sp: (sp[0, g], 0, 0)),
        ],
        out_specs=pl.BlockSpec((_TILE, d), lambda g, sp: (g, 0)),
    )
    return pl.pallas_call(
        _ffn_body,
        grid_spec=grid_spec,
        out_shape=jax.ShapeDtypeStruct((pad_rows, d), jnp.float32),
        interpret=_INTERPRET,
    )(
        sp,
        xs,
        row_gate.reshape(n_blocks, 1, _TILE),
        w1,
        b1.reshape(_E, 1, 2 * ff),
        w2,
        b2.reshape(_E, 1, d),
    )


def kernel(x, Wr, w1, b1, w2, b2):
    bx, tx, d = x.shape
    n = bx * tx
    s = n * _K
    n_blocks = s // _TILE + _E
    flat_x = x.reshape(n, d)

    gates, idx = _router(flat_x, Wr)
    row_token, row_gate, pos, sp = _build_dispatch(idx, gates, n_blocks)
    xs = flat_x[row_token]
    ys = _grouped_ffn(xs, row_gate, w1, b1, w2, b2, sp, n_blocks)
    out = ys[pos[:, 0]] + ys[pos[:, 1]]
    return out.reshape(bx, tx, d)


# speedup vs baseline: 2.2449x; 1.3371x over previous
import functools

import jax
import jax.numpy as jnp
from jax import lax
from jax.experimental import pallas as pl
from jax.experimental.pallas import tpu as pltpu
from jax.experimental.pallas import tpu_sc as plsc

_E = 8
_K = 2
_TILE = 256


def _router_body(x_ref, wr_ref, gates_ref, idx_ref):
    logits = jnp.dot(x_ref[...], wr_ref[...], preferred_element_type=jnp.float32)
    ei = lax.broadcasted_iota(jnp.int32, logits.shape, 1)
    m1 = jnp.max(logits, axis=1, keepdims=True)
    i1 = jnp.min(jnp.where(logits == m1, ei, _E), axis=1, keepdims=True)
    masked = jnp.where(ei == i1, -jnp.inf, logits)
    m2 = jnp.max(masked, axis=1, keepdims=True)
    i2 = jnp.min(jnp.where(masked == m2, ei, _E), axis=1, keepdims=True)
    z = jnp.exp(m2 - m1)
    g1 = 1.0 / (1.0 + z)
    gates_ref[...] = jnp.concatenate([g1, 1.0 - g1], axis=1)
    idx_ref[...] = jnp.concatenate([i1, i2], axis=1).astype(jnp.int32)


def _router(flat_x, Wr):
    n = flat_x.shape[0]
    return pl.pallas_call(
        _router_body,
        out_shape=[
            jax.ShapeDtypeStruct((n, _K), jnp.float32),
            jax.ShapeDtypeStruct((n, _K), jnp.int32),
        ],
    )(flat_x, Wr)


def _sc_dispatch(flat_x, ef, n_blocks):
    n, d = flat_x.shape
    s = ef.shape[0]
    info = plsc.get_sparse_core_info()
    nc = info.num_cores
    nw = nc * info.num_subcores
    chunk = s // nw
    nvec = s // 16
    vper = chunk // 16
    cw = 64
    pad_rows = n_blocks * _TILE

    @functools.partial(
        pl.kernel,
        mesh=plsc.VectorSubcoreMesh(core_axis_name="c", subcore_axis_name="s"),
        compiler_params=pltpu.CompilerParams(needs_layout_passes=False),
        out_type=[
            jax.ShapeDtypeStruct((pad_rows, d), jnp.float32),
            jax.ShapeDtypeStruct((nw, vper // 4, 4 * 16), jnp.int32),
            jax.ShapeDtypeStruct((2, 48), jnp.int32),
        ],
        scratch_types=[
            pltpu.VMEM((s,), jnp.int32),
            pltpu.VMEM((vper // 4, 4 * 16), jnp.int32),
            pltpu.VMEM((vper // 4, 4 * 16), jnp.int32),
            pltpu.VMEM((cw, d), jnp.float32),
            pltpu.VMEM((2, 48), jnp.int32),
            pltpu.SemaphoreType.DMA,
        ],
    )
    def k(x_hbm, ef_hbm, xs_hbm, pos_hbm, sp_hbm, ef_v, tok_v, p_v, rows_v,
          sp_v, sem):
        wid = lax.axis_index("s") * nc + lax.axis_index("c")
        zero = jnp.zeros((16,), jnp.int32)
        one = jnp.ones((16,), jnp.int32)
        pltpu.sync_copy(ef_hbm, ef_v)
        my0 = wid * vper

        def mk_body(accum_pre):
            def body(i, c):
                ev = ef_v[pl.ds(i * 16, 16)]
                tots = list(c[:_E])
                pres = list(c[_E:])
                for e in range(_E):
                    mi = jnp.where(ev == e, one, zero)
                    tots[e] = tots[e] + mi
                    if accum_pre:
                        pres[e] = pres[e] + mi
                return tuple(tots) + tuple(pres)

            return body

        init = (zero,) * (2 * _E)
        c1 = lax.fori_loop(0, my0, mk_body(True), init)
        c2 = lax.fori_loop(my0, nvec, mk_body(False), c1)
        tot = [jnp.sum(v) for v in c2[:_E]]
        pre = [jnp.sum(v) for v in c2[_E:]]

        starts = []
        cbs = []
        cb = jnp.int32(0)
        for e in range(_E):
            nblk = (tot[e] + (_TILE - 1)) // _TILE
            starts.append(cb * _TILE + pre[e])
            cb = cb + nblk
            cbs.append(cb)

        lane = lax.iota(jnp.int32, 16)
        base_s = wid * chunk
        for kk in range(vper):
            ev = ef_v[pl.ds(base_s + kk * 16, 16)]
            p = zero
            for e in range(_E):
                m = ev == e
                mi = jnp.where(m, one, zero)
                cs = plsc.cumsum(mi)
                p = p + jnp.where(m, starts[e] + cs - 1, zero)
                starts[e] = starts[e] + jnp.sum(mi)
            tok = (base_s + kk * 16 + lane) // _K
            p_v[kk // 4, pl.ds((kk % 4) * 16, 16)] = p
            tok_v[kk // 4, pl.ds((kk % 4) * 16, 16)] = tok
        pltpu.sync_copy(p_v, pos_hbm.at[wid])

        @pl.when(wid == 0)
        def _():
            for j in range(3):
                gv = lane + j * 16
                be = zero
                for e in range(_E):
                    be = be + jnp.where(gv >= cbs[e], one, zero)
                sp_v[0, pl.ds(j * 16, 16)] = jnp.minimum(be, _E - 1)
                sp_v[1, pl.ds(j * 16, 16)] = be
            pltpu.sync_copy(sp_v, sp_hbm)

        for j in range(chunk // cw):
            pltpu.async_copy(x_hbm.at[tok_v.at[j]], rows_v, sem).wait()
            pltpu.async_copy(rows_v, xs_hbm.at[p_v.at[j]], sem).wait()

    return k(flat_x, ef)


def _ffn_body(sp_ref, xs_ref, w1_ref, b1_ref, w2_ref, b2_ref, ys_ref):
    ff = w2_ref.shape[1]

    @pl.when(sp_ref[1, pl.program_id(0)] < _E)
    def _():
        u = jnp.dot(xs_ref[...], w1_ref[0], preferred_element_type=jnp.float32)
        u = u + b1_ref[0]
        ua = u[:, :ff]
        ub = u[:, ff:]
        h = ua * jax.nn.sigmoid(ua) * ub
        y = jnp.dot(h, w2_ref[0], preferred_element_type=jnp.float32)
        ys_ref[...] = y + b2_ref[0]


def _grouped_ffn(xs, w1, b1, w2, b2, sp, n_blocks):
    d = xs.shape[1]
    ff = w2.shape[1]
    pad_rows = n_blocks * _TILE
    grid_spec = pltpu.PrefetchScalarGridSpec(
        num_scalar_prefetch=1,
        grid=(n_blocks,),
        in_specs=[
            pl.BlockSpec((_TILE, d), lambda g, sp: (g, 0)),
            pl.BlockSpec((1, d, 2 * ff), lambda g, sp: (sp[0, g], 0, 0)),
            pl.BlockSpec((1, 1, 2 * ff), lambda g, sp: (sp[0, g], 0, 0)),
            pl.BlockSpec((1, ff, d), lambda g, sp: (sp[0, g], 0, 0)),
            pl.BlockSpec((1, 1, d), lambda g, sp: (sp[0, g], 0, 0)),
        ],
        out_specs=pl.BlockSpec((_TILE, d), lambda g, sp: (g, 0)),
    )
    return pl.pallas_call(
        _ffn_body,
        grid_spec=grid_spec,
        out_shape=jax.ShapeDtypeStruct((pad_rows, d), jnp.float32),
    )(
        sp,
        xs,
        w1,
        b1.reshape(_E, 1, 2 * ff),
        w2,
        b2.reshape(_E, 1, d),
    )


def kernel(x, Wr, w1, b1, w2, b2):
    bx, tx, d = x.shape
    n = bx * tx
    s = n * _K
    n_blocks = s // _TILE + _E
    flat_x = x.reshape(n, d)

    gates, idx = _router(flat_x, Wr)
    xs, pos_raw, sp = _sc_dispatch(flat_x, idx.reshape(-1), n_blocks)
    ys = _grouped_ffn(xs, w1, b1, w2, b2, sp, n_blocks)
    pos = pos_raw.reshape(n, _K)
    out = gates[:, :1] * ys[pos[:, 0]] + gates[:, 1:] * ys[pos[:, 1]]
    return out.reshape(bx, tx, d)
